# baseline (speedup 1.0000x reference)
import dataclasses
import functools

import jax
import jax.numpy as jnp
from jax import lax
from jax.experimental import pallas as pl
from jax.experimental.pallas import tpu as pltpu
from jax.experimental.pallas import tpu_sc as plsc

_D = 128
_W = 80
_NC = 2
_NS = 16
_NW = _NC * _NS
_L = 16
_LB = 32
_NSLOT = 4


def _sc_dot_gather(z_u32, edge_index, n_edges):
    epw = n_edges // _NW
    n_ch = -(-epw // _W)
    epw_pad = n_ch * _W
    mesh = plsc.VectorSubcoreMesh(core_axis_name="c", subcore_axis_name="s")
    cp = pltpu.CompilerParams()
    if "needs_layout_passes" in pltpu.CompilerParams.__dataclass_fields__:
        cp = dataclasses.replace(cp, needs_layout_passes=False)
    if "use_tc_tiling_on_sc" in pltpu.CompilerParams.__dataclass_fields__:
        cp = dataclasses.replace(cp, use_tc_tiling_on_sc=False)

    @functools.partial(
        pl.kernel,
        out_type=jax.ShapeDtypeStruct((n_edges,), jnp.float32),
        mesh=mesh,
        compiler_params=cp,
        scratch_types=[
            pltpu.VMEM((epw_pad,), jnp.int32),
            pltpu.VMEM((epw_pad,), jnp.int32),
            *([pltpu.VMEM((_W, _D // 2), jnp.uint32)] * (2 * _NSLOT)),
            pltpu.VMEM((_W * _L,), jnp.float32),
            pltpu.VMEM((epw_pad,), jnp.float32),
            *([pltpu.SemaphoreType.DMA] * _NSLOT),
            pltpu.SemaphoreType.DMA,
        ],
    )
    def k(z_hbm, e_hbm, out_hbm, sidx, didx, *rest):
        srow_bufs = rest[:_NSLOT]
        drow_bufs = rest[_NSLOT:2 * _NSLOT]
        part, outa = rest[2 * _NSLOT], rest[2 * _NSLOT + 1]
        gsems = rest[2 * _NSLOT + 2:3 * _NSLOT + 2]
        isem = rest[3 * _NSLOT + 2]
        wid = lax.axis_index("s") * _NC + lax.axis_index("c")
        base0 = wid * epw
        coloffs = lax.iota(jnp.int32, _L) * _L
        zeros = jnp.zeros((_L,), jnp.int32)

        c1 = pltpu.async_copy(e_hbm.at[0, pl.ds(base0, epw)],
                              sidx.at[pl.ds(0, epw)], isem)
        c2 = pltpu.async_copy(e_hbm.at[1, pl.ds(base0, epw)],
                              didx.at[pl.ds(0, epw)], isem)
        for t in range(epw, epw_pad, _L):
            sidx[pl.ds(t, _L)] = zeros
            didx[pl.ds(t, _L)] = zeros
        c1.wait()
        c2.wait()

        def issue(c, sbuf, dbuf, sem):
            off = c * _W
            pltpu.async_copy(z_hbm.at[sidx.at[pl.ds(off, _W)]], sbuf, sem)
            pltpu.async_copy(z_hbm.at[didx.at[pl.ds(off, _W)]], dbuf, sem)

        def drain(c, sbuf, dbuf, sem):
            off = c * _W
            pltpu.make_async_copy(
                z_hbm.at[sidx.at[pl.ds(off, _W)]], sbuf, sem).wait()
            pltpu.make_async_copy(
                z_hbm.at[didx.at[pl.ds(off, _W)]], dbuf, sem).wait()

        def compute(c, sbuf, dbuf):
            @plsc.parallel_loop(0, _W, unroll=2)
            def _(r):
                acc = None
                for kk in range(_D // _LB):
                    sv = plsc.bitcast(sbuf[r, pl.ds(kk * _L, _L)],
                                      jnp.bfloat16)
                    dv = plsc.bitcast(dbuf[r, pl.ds(kk * _L, _L)],
                                      jnp.bfloat16)
                    p = sv * dv
                    lo, hi = plsc.unpack(p, format=plsc.PackFormat.INTERLEAVED)
                    s = lo + hi
                    acc = s if acc is None else acc + s
                part[pl.ds(r * _L, _L)] = acc

            @plsc.parallel_loop(0, _W // _L, unroll=1)
            def _(g):
                red = plsc.load_gather(part, [coloffs + g * (_L * _L)])
                for kk in range(1, _L):
                    red = red + plsc.load_gather(
                        part, [coloffs + (g * (_L * _L) + kk)])
                outa[pl.ds(c * _W + g * _L, _L)] = red

        slots = tuple(zip(srow_bufs, drow_bufs, gsems))
        n_slots = len(slots)
        n_pad = -(-n_ch // n_slots) * n_slots

        for b in range(n_slots - 1):
            issue(b, *slots[b])

        @pl.loop(0, n_pad, step=n_slots)
        def _(i):
            for b in range(n_slots):
                sbuf, dbuf, sem = slots[b]
                c = i + b
                nxt = c + (n_slots - 1)

                @pl.when(nxt <= n_ch - 1)
                def _():
                    sb, db, sm = slots[(b + n_slots - 1) % n_slots]
                    issue(nxt, sb, db, sm)

                @pl.when(c <= n_ch - 1)
                def _():
                    drain(c, sbuf, dbuf, sem)
                    compute(c, sbuf, dbuf)

        pltpu.sync_copy(outa.at[pl.ds(0, epw)], out_hbm.at[pl.ds(base0, epw)])

    return k(z_u32, edge_index)


def kernel(z, edge_index):
    n_edges = edge_index.shape[1]
    zu = jax.lax.bitcast_convert_type(z, jnp.uint32)

    def _rnd(x):
        return (x + jnp.uint32(0x7FFF) + ((x >> 16) & jnp.uint32(1))) >> 16

    z_u32 = _rnd(zu[:, 0::2]) | (_rnd(zu[:, 1::2]) << 16)
    return _sc_dot_gather(z_u32, edge_index.astype(jnp.int32), n_edges)


# speedup vs baseline: 2.8889x; 2.8889x over previous
import dataclasses
import functools

import jax
import jax.numpy as jnp
from jax import lax
from jax.experimental import pallas as pl
from jax.experimental.pallas import tpu as pltpu
from jax.experimental.pallas import tpu_sc as plsc

_D = 128
_W = 80
_NC = 2
_NS = 16
_NW = _NC * _NS
_L = 16
_LB = 32
_NSLOT = 4


def _sc_dot_gather(z_u32, edge_index, n_edges):
    epw = n_edges // _NW
    n_ch = -(-epw // _W)
    epw_pad = n_ch * _W
    mesh = plsc.VectorSubcoreMesh(core_axis_name="c", subcore_axis_name="s")
    cp = pltpu.CompilerParams()
    if "needs_layout_passes" in pltpu.CompilerParams.__dataclass_fields__:
        cp = dataclasses.replace(cp, needs_layout_passes=False)
    if "use_tc_tiling_on_sc" in pltpu.CompilerParams.__dataclass_fields__:
        cp = dataclasses.replace(cp, use_tc_tiling_on_sc=False)

    @functools.partial(
        pl.kernel,
        out_type=jax.ShapeDtypeStruct((n_edges,), jnp.float32),
        mesh=mesh,
        compiler_params=cp,
        scratch_types=[
            pltpu.VMEM((epw_pad,), jnp.int32),
            pltpu.VMEM((epw_pad,), jnp.int32),
            *([pltpu.VMEM((_W, _D // 2), jnp.uint32)] * (2 * _NSLOT)),
            pltpu.VMEM((_W * _L,), jnp.float32),
            pltpu.VMEM((epw_pad,), jnp.float32),
            *([pltpu.SemaphoreType.DMA] * _NSLOT),
            pltpu.SemaphoreType.DMA,
        ],
    )
    def k(z_hbm, e_hbm, out_hbm, sidx, didx, *rest):
        srow_bufs = rest[:_NSLOT]
        drow_bufs = rest[_NSLOT:2 * _NSLOT]
        part, outa = rest[2 * _NSLOT], rest[2 * _NSLOT + 1]
        gsems = rest[2 * _NSLOT + 2:3 * _NSLOT + 2]
        isem = rest[3 * _NSLOT + 2]
        wid = lax.axis_index("s") * _NC + lax.axis_index("c")
        base0 = wid * epw
        coloffs = lax.iota(jnp.int32, _L) * _L
        zeros = jnp.zeros((_L,), jnp.int32)

        c1 = pltpu.async_copy(e_hbm.at[0, pl.ds(base0, epw)],
                              sidx.at[pl.ds(0, epw)], isem)
        c2 = pltpu.async_copy(e_hbm.at[1, pl.ds(base0, epw)],
                              didx.at[pl.ds(0, epw)], isem)
        for t in range(epw, epw_pad, _L):
            sidx[pl.ds(t, _L)] = zeros
            didx[pl.ds(t, _L)] = zeros
        c1.wait()
        c2.wait()

        def issue(c, sbuf, dbuf, sem):
            off = c * _W
            pltpu.async_copy(z_hbm.at[sidx.at[pl.ds(off, _W)]], sbuf, sem)
            pltpu.async_copy(z_hbm.at[didx.at[pl.ds(off, _W)]], dbuf, sem)

        def drain(c, sbuf, dbuf, sem):
            off = c * _W
            pltpu.make_async_copy(
                z_hbm.at[sidx.at[pl.ds(off, _W)]], sbuf, sem).wait()
            pltpu.make_async_copy(
                z_hbm.at[didx.at[pl.ds(off, _W)]], dbuf, sem).wait()

        def compute(c, sbuf, dbuf):
            @plsc.parallel_loop(0, _W, unroll=2)
            def _(r):
                acc = None
                for kk in range(_D // _LB):
                    sv = plsc.bitcast(sbuf[r, pl.ds(kk * _L, _L)],
                                      jnp.bfloat16)
                    dv = plsc.bitcast(dbuf[r, pl.ds(kk * _L, _L)],
                                      jnp.bfloat16)
                    p = sv * dv
                    lo, hi = plsc.unpack(p, format=plsc.PackFormat.INTERLEAVED)
                    s = lo + hi
                    acc = s if acc is None else acc + s
                part[pl.ds(r * _L, _L)] = acc

            @plsc.parallel_loop(0, _W // _L, unroll=1)
            def _(g):
                red = plsc.load_gather(part, [coloffs + g * (_L * _L)])
                for kk in range(1, _L):
                    red = red + plsc.load_gather(
                        part, [coloffs + (g * (_L * _L) + kk)])
                outa[pl.ds(c * _W + g * _L, _L)] = red

        slots = tuple(zip(srow_bufs, drow_bufs, gsems))
        n_slots = len(slots)
        n_pad = -(-n_ch // n_slots) * n_slots

        for b in range(n_slots - 1):
            issue(b, *slots[b])

        @pl.loop(0, n_pad, step=n_slots)
        def _(i):
            for b in range(n_slots):
                sbuf, dbuf, sem = slots[b]
                c = i + b
                nxt = c + (n_slots - 1)

                @pl.when(nxt <= n_ch - 1)
                def _():
                    sb, db, sm = slots[(b + n_slots - 1) % n_slots]
                    issue(nxt, sb, db, sm)

                @pl.when(c <= n_ch - 1)
                def _():
                    drain(c, sbuf, dbuf, sem)
                    compute(c, sbuf, dbuf)

        pltpu.sync_copy(outa.at[pl.ds(0, epw)], out_hbm.at[pl.ds(base0, epw)])

    return k(z_u32, edge_index)


def kernel(z, edge_index):
    n_edges = edge_index.shape[1]
    zu = jax.lax.bitcast_convert_type(z, jnp.uint32)

    def _rnd(x):
        return (x + jnp.uint32(0x7FFF) + ((x >> 16) & jnp.uint32(1))) >> 16

    half = z.shape[1] // 2
    z_u32 = _rnd(zu[:, :half]) | (_rnd(zu[:, half:]) << 16)
    return _sc_dot_gather(z_u32, edge_index.astype(jnp.int32), n_edges)


# speedup vs baseline: 2.8927x; 1.0013x over previous
import dataclasses
import functools

import jax
import jax.numpy as jnp
from jax import lax
from jax.experimental import pallas as pl
from jax.experimental.pallas import tpu as pltpu
from jax.experimental.pallas import tpu_sc as plsc

_D = 128
_W = 80
_NC = 2
_NS = 16
_NW = _NC * _NS
_L = 16
_LB = 32
_NSLOT = 5


def _sc_dot_gather(z_u32, edge_index, n_edges):
    epw = n_edges // _NW
    n_ch = -(-epw // _W)
    epw_pad = n_ch * _W
    mesh = plsc.VectorSubcoreMesh(core_axis_name="c", subcore_axis_name="s")
    cp = pltpu.CompilerParams()
    if "needs_layout_passes" in pltpu.CompilerParams.__dataclass_fields__:
        cp = dataclasses.replace(cp, needs_layout_passes=False)
    if "use_tc_tiling_on_sc" in pltpu.CompilerParams.__dataclass_fields__:
        cp = dataclasses.replace(cp, use_tc_tiling_on_sc=False)

    @functools.partial(
        pl.kernel,
        out_type=jax.ShapeDtypeStruct((n_edges,), jnp.float32),
        mesh=mesh,
        compiler_params=cp,
        scratch_types=[
            pltpu.VMEM((epw_pad,), jnp.int32),
            pltpu.VMEM((epw_pad,), jnp.int32),
            *([pltpu.VMEM((_W, _D // 2), jnp.uint32)] * (2 * _NSLOT)),
            pltpu.VMEM((_W * _L,), jnp.float32),
            pltpu.VMEM((epw_pad,), jnp.float32),
            *([pltpu.SemaphoreType.DMA] * _NSLOT),
            pltpu.SemaphoreType.DMA,
        ],
    )
    def k(z_hbm, e_hbm, out_hbm, sidx, didx, *rest):
        srow_bufs = rest[:_NSLOT]
        drow_bufs = rest[_NSLOT:2 * _NSLOT]
        part, outa = rest[2 * _NSLOT], rest[2 * _NSLOT + 1]
        gsems = rest[2 * _NSLOT + 2:3 * _NSLOT + 2]
        isem = rest[3 * _NSLOT + 2]
        wid = lax.axis_index("s") * _NC + lax.axis_index("c")
        base0 = wid * epw
        coloffs = lax.iota(jnp.int32, _L) * _L
        zeros = jnp.zeros((_L,), jnp.int32)

        c1 = pltpu.async_copy(e_hbm.at[0, pl.ds(base0, epw)],
                              sidx.at[pl.ds(0, epw)], isem)
        c2 = pltpu.async_copy(e_hbm.at[1, pl.ds(base0, epw)],
                              didx.at[pl.ds(0, epw)], isem)
        for t in range(epw, epw_pad, _L):
            sidx[pl.ds(t, _L)] = zeros
            didx[pl.ds(t, _L)] = zeros
        c1.wait()
        c2.wait()

        def issue(c, sbuf, dbuf, sem):
            off = c * _W
            pltpu.async_copy(z_hbm.at[sidx.at[pl.ds(off, _W)]], sbuf, sem)
            pltpu.async_copy(z_hbm.at[didx.at[pl.ds(off, _W)]], dbuf, sem)

        def drain(c, sbuf, dbuf, sem):
            off = c * _W
            pltpu.make_async_copy(
                z_hbm.at[sidx.at[pl.ds(off, _W)]], sbuf, sem).wait()
            pltpu.make_async_copy(
                z_hbm.at[didx.at[pl.ds(off, _W)]], dbuf, sem).wait()

        def compute(c, sbuf, dbuf):
            @plsc.parallel_loop(0, _W, unroll=2)
            def _(r):
                acc = None
                for kk in range(_D // _LB):
                    sv = plsc.bitcast(sbuf[r, pl.ds(kk * _L, _L)],
                                      jnp.bfloat16)
                    dv = plsc.bitcast(dbuf[r, pl.ds(kk * _L, _L)],
                                      jnp.bfloat16)
                    p = sv * dv
                    lo, hi = plsc.unpack(p, format=plsc.PackFormat.INTERLEAVED)
                    s = lo + hi
                    acc = s if acc is None else acc + s
                part[pl.ds(r * _L, _L)] = acc

            @plsc.parallel_loop(0, _W // _L, unroll=1)
            def _(g):
                red = plsc.load_gather(part, [coloffs + g * (_L * _L)])
                for kk in range(1, _L):
                    red = red + plsc.load_gather(
                        part, [coloffs + (g * (_L * _L) + kk)])
                outa[pl.ds(c * _W + g * _L, _L)] = red

        slots = tuple(zip(srow_bufs, drow_bufs, gsems))
        n_slots = len(slots)
        n_pad = -(-n_ch // n_slots) * n_slots

        for b in range(n_slots - 1):
            issue(b, *slots[b])

        @pl.loop(0, n_pad, step=n_slots)
        def _(i):
            for b in range(n_slots):
                sbuf, dbuf, sem = slots[b]
                c = i + b
                nxt = c + (n_slots - 1)

                @pl.when(nxt <= n_ch - 1)
                def _():
                    sb, db, sm = slots[(b + n_slots - 1) % n_slots]
                    issue(nxt, sb, db, sm)

                @pl.when(c <= n_ch - 1)
                def _():
                    drain(c, sbuf, dbuf, sem)
                    compute(c, sbuf, dbuf)

        pltpu.sync_copy(outa.at[pl.ds(0, epw)], out_hbm.at[pl.ds(base0, epw)])

    return k(z_u32, edge_index)


def kernel(z, edge_index):
    n_edges = edge_index.shape[1]
    zu = jax.lax.bitcast_convert_type(z, jnp.uint32)

    def _rnd(x):
        return (x + jnp.uint32(0x7FFF) + ((x >> 16) & jnp.uint32(1))) >> 16

    half = z.shape[1] // 2
    z_u32 = _rnd(zu[:, :half]) | (_rnd(zu[:, half:]) << 16)
    return _sc_dot_gather(z_u32, edge_index.astype(jnp.int32), n_edges)
